# baseline (speedup 1.0000x reference)
import functools
import math

import jax
import jax.numpy as jnp
from jax import lax
from jax.experimental import pallas as pl
from jax.experimental.pallas import tpu as pltpu
from jax.experimental.pallas import tpu_sc as plsc

N_NODES = 10000
N_EDGES = 320000
DIM = 128
AVG_NEIGH = 32.0

NC = 2
NS = 16
NW = NC * NS
CHUNK = 128
NBUF = 2
SUPER = 16
CH_PER_W = 80
E_PAD = NW * CH_PER_W * CHUNK
ROWS_PER_TILE = 624


def _node_body(x_ref, na_ref, w1_ref, wsc_ref, h_ref, sc_ref):
    x = x_ref[...]
    na = na_ref[...]
    h_ref[...] = (x @ w1_ref[...]) * jnp.float32(1.0 / math.sqrt(AVG_NEIGH))
    acc = (x * na[:, 0:1]) @ wsc_ref[0]
    for a in range(1, 4):
        acc = acc + (x * na[:, a:a + 1]) @ wsc_ref[a]
    sc_ref[...] = acc


def _node_tc(x, node_attrs, w1, w_sc):
    blk = 1000
    grid = (N_NODES // blk,)
    return pl.pallas_call(
        _node_body,
        grid=grid,
        in_specs=[
            pl.BlockSpec((blk, DIM), lambda i: (i, 0)),
            pl.BlockSpec((blk, 4), lambda i: (i, 0)),
            pl.BlockSpec((DIM, DIM), lambda i: (0, 0)),
            pl.BlockSpec((4, DIM, DIM), lambda i: (0, 0, 0)),
        ],
        out_specs=[
            pl.BlockSpec((blk, DIM), lambda i: (i, 0)),
            pl.BlockSpec((blk, DIM), lambda i: (i, 0)),
        ],
        out_shape=[
            jax.ShapeDtypeStruct((N_NODES, DIM), jnp.float32),
            jax.ShapeDtypeStruct((N_NODES, DIM), jnp.float32),
        ],
    )(x, node_attrs, w1, w_sc)


def _edge_body(eet_ref, eat_ref, wm1t_ref, bm1_ref, wm2t_ref, bm2_ref, scal_ref):
    i = pl.program_id(0)
    blk = eet_ref.shape[1]
    z = wm1t_ref[...] @ eet_ref[...] + bm1_ref[...]
    z = z * jax.nn.sigmoid(z)
    w = wm2t_ref[...] @ z + bm2_ref[...]
    scal_ref[pl.ds(i * blk, blk)] = jnp.sum(w * eat_ref[...], axis=0)


def _edge_tc(eet, eat, wm1t, bm1c, wm2t, bm2c):
    blk = 16000
    grid = (N_EDGES // blk,)
    s_dim = eet.shape[0]
    h_dim = wm1t.shape[0]
    return pl.pallas_call(
        _edge_body,
        grid=grid,
        in_specs=[
            pl.BlockSpec((s_dim, blk), lambda i: (0, i)),
            pl.BlockSpec((s_dim, blk), lambda i: (0, i)),
            pl.BlockSpec((h_dim, s_dim), lambda i: (0, 0)),
            pl.BlockSpec((h_dim, 1), lambda i: (0, 0)),
            pl.BlockSpec((s_dim, h_dim), lambda i: (0, 0)),
            pl.BlockSpec((s_dim, 1), lambda i: (0, 0)),
        ],
        out_specs=pl.BlockSpec((N_EDGES,), lambda i: (0,)),
        out_shape=jax.ShapeDtypeStruct((N_EDGES,), jnp.float32),
    )(eet, eat, wm1t, bm1c, wm2t, bm2c)


def _sc_body(h_hbm, src_hbm, dst_hbm, scal_hbm, out_hbm,
             src_sb, dst_sb, scal_sb, rows, agg, gsem, ssem):
    c = lax.axis_index("c")
    s = lax.axis_index("s")
    wid = c * NS + s

    zero16 = jnp.zeros((16,), jnp.float32)

    @pl.loop(0, CHUNK)
    def _zero_rows(i):
        for d in range(DIM // 16):
            rows[0, i, pl.ds(d * 16, 16)] = zero16

    base = s * ROWS_PER_TILE
    off = 0
    for cnt in (128, 128, 128, 128, 112):
        pltpu.sync_copy(rows.at[0, pl.ds(0, cnt)], agg.at[pl.ds(base + off, cnt)])
        off += cnt

    @pl.when(s == NS - 1)
    def _zero_tail():
        pltpu.sync_copy(rows.at[0, pl.ds(0, 16)],
                        agg.at[pl.ds(NS * ROWS_PER_TILE, 16)])

    row0 = wid * CH_PER_W
    plsc.subcore_barrier()

    @pl.loop(0, CH_PER_W // SUPER)
    def _super(k):
        r0 = row0 + k * SUPER
        pltpu.sync_copy(src_hbm.at[pl.ds(r0, SUPER)], src_sb)
        pltpu.sync_copy(dst_hbm.at[pl.ds(r0, SUPER)], dst_sb)
        pltpu.sync_copy(scal_hbm.at[pl.ds(r0 * CHUNK, SUPER * CHUNK)], scal_sb)

        for b in range(NBUF):
            pltpu.async_copy(h_hbm.at[src_sb.at[b]], rows.at[b], gsem.at[b])

        @pl.loop(0, SUPER // NBUF)
        def _ring(t):
            for b in range(NBUF):
                j = t * NBUF + b
                bprev = (b - 1) % NBUF
                pltpu.make_async_copy(h_hbm.at[src_sb.at[0]], rows.at[b],
                                      gsem.at[b]).wait()

                @pl.loop(0, CHUNK, unroll=8)
                def _scale(e):
                    sv = plsc.load_gather(
                        scal_sb, [jnp.full((16,), j * CHUNK + e, jnp.int32)])
                    for d in range(DIM // 16):
                        rows[b, e, pl.ds(d * 16, 16)] = (
                            rows[b, e, pl.ds(d * 16, 16)] * sv)

                pltpu.async_copy(rows.at[b], agg.at[dst_sb.at[j]], ssem.at[b],
                                 add=True)

                @pl.when(j > 0)
                def _retire():
                    pltpu.make_async_copy(rows.at[bprev],
                                          agg.at[dst_sb.at[0]],
                                          ssem.at[bprev]).wait()

                    @pl.when(j + NBUF - 1 < SUPER)
                    def _refill():
                        pltpu.async_copy(h_hbm.at[src_sb.at[j + NBUF - 1]],
                                         rows.at[bprev], gsem.at[bprev])

        pltpu.make_async_copy(rows.at[NBUF - 1], agg.at[dst_sb.at[0]],
                              ssem.at[NBUF - 1]).wait()

    plsc.subcore_barrier()
    pltpu.sync_copy(agg.at[pl.ds(base, ROWS_PER_TILE)],
                    out_hbm.at[c, pl.ds(base, ROWS_PER_TILE)])

    @pl.when(s == NS - 1)
    def _copy_tail():
        pltpu.sync_copy(agg.at[pl.ds(NS * ROWS_PER_TILE, 16)],
                        out_hbm.at[c, pl.ds(NS * ROWS_PER_TILE, 16)])


def _sc_scatter(h, src2d, dst2d, scal2d):
    mesh = plsc.VectorSubcoreMesh(core_axis_name="c", subcore_axis_name="s")
    k = pl.kernel(
        _sc_body,
        out_type=jax.ShapeDtypeStruct((NC, N_NODES, DIM), jnp.float32),
        mesh=mesh,
        scratch_types=[
            pltpu.VMEM((SUPER, CHUNK), jnp.int32),
            pltpu.VMEM((SUPER, CHUNK), jnp.int32),
            pltpu.VMEM((SUPER * CHUNK,), jnp.float32),
            pltpu.VMEM((NBUF, CHUNK, DIM), jnp.float32),
            pltpu.VMEM_SHARED((N_NODES, DIM), jnp.float32),
            pltpu.SemaphoreType.DMA((NBUF,)),
            pltpu.SemaphoreType.DMA((NBUF,)),
        ],
        compiler_params=pltpu.CompilerParams(needs_layout_passes=False),
    )
    return k(h, src2d, dst2d, scal2d)


def _out_body(agg_ref, sc_ref, w2_ref, out_ref):
    a = agg_ref[0] + agg_ref[1]
    y = a @ w2_ref[...]
    out_ref[...] = y * jax.nn.sigmoid(y) + sc_ref[...]


def _out_tc(agg, sc, w2):
    blk = 1000
    grid = (N_NODES // blk,)
    return pl.pallas_call(
        _out_body,
        grid=grid,
        in_specs=[
            pl.BlockSpec((NC, blk, DIM), lambda i: (0, i, 0)),
            pl.BlockSpec((blk, DIM), lambda i: (i, 0)),
            pl.BlockSpec((DIM, DIM), lambda i: (0, 0)),
        ],
        out_specs=pl.BlockSpec((blk, DIM), lambda i: (i, 0)),
        out_shape=jax.ShapeDtypeStruct((N_NODES, DIM), jnp.float32),
    )(agg, sc, w2)


def kernel(x, node_attrs, edge_embedding, edge_attr, edge_index,
           W1, W_sc, Wm1, bm1, Wm2, bm2, W2):
    h, sc = _node_tc(x, node_attrs, W1, W_sc)
    scal = _edge_tc(edge_embedding.T, edge_attr.T, Wm1.T, bm1[:, None],
                    Wm2.T, bm2[:, None])

    pad = E_PAD - N_EDGES
    dst = edge_index[0]
    src = edge_index[1]
    pi = jnp.arange(pad, dtype=jnp.int32) % N_NODES
    src2d = jnp.concatenate([src, pi]).reshape(E_PAD // CHUNK, CHUNK)
    dst2d = jnp.concatenate([dst, pi]).reshape(E_PAD // CHUNK, CHUNK)
    scal_p = jnp.concatenate([scal, jnp.zeros((pad,), jnp.float32)])

    agg = _sc_scatter(h, src2d, dst2d, scal_p)
    return _out_tc(agg, sc, W2)


# speedup vs baseline: 1.1574x; 1.1574x over previous
import functools
import math

import jax
import jax.numpy as jnp
from jax import lax
from jax.experimental import pallas as pl
from jax.experimental.pallas import tpu as pltpu
from jax.experimental.pallas import tpu_sc as plsc

N_NODES = 10000
N_EDGES = 320000
DIM = 128
AVG_NEIGH = 32.0

NC = 2
NS = 16
NW = NC * NS
CHUNK = 128
NBUF = 2
SUPER = 16
CH_PER_W = 80
E_PAD = NW * CH_PER_W * CHUNK
ROWS_PER_TILE = 624


def _node_body(x_ref, na_ref, w1_ref, wsc_ref, h_ref, sc_ref):
    x = x_ref[...]
    na = na_ref[...]
    h_ref[...] = (x @ w1_ref[...]) * jnp.float32(1.0 / math.sqrt(AVG_NEIGH))
    acc = (x * na[:, 0:1]) @ wsc_ref[0]
    for a in range(1, 4):
        acc = acc + (x * na[:, a:a + 1]) @ wsc_ref[a]
    sc_ref[...] = acc


def _node_tc(x, node_attrs, w1, w_sc):
    blk = 1000
    grid = (N_NODES // blk,)
    return pl.pallas_call(
        _node_body,
        grid=grid,
        in_specs=[
            pl.BlockSpec((blk, DIM), lambda i: (i, 0)),
            pl.BlockSpec((blk, 4), lambda i: (i, 0)),
            pl.BlockSpec((DIM, DIM), lambda i: (0, 0)),
            pl.BlockSpec((4, DIM, DIM), lambda i: (0, 0, 0)),
        ],
        out_specs=[
            pl.BlockSpec((blk, DIM), lambda i: (i, 0)),
            pl.BlockSpec((blk, DIM), lambda i: (i, 0)),
        ],
        out_shape=[
            jax.ShapeDtypeStruct((N_NODES, DIM), jnp.float32),
            jax.ShapeDtypeStruct((N_NODES, DIM), jnp.float32),
        ],
    )(x, node_attrs, w1, w_sc)


def _edge_body(eet_ref, eat_ref, wm1t_ref, bm1_ref, wm2t_ref, bm2_ref, scal_ref):
    i = pl.program_id(0)
    blk = eet_ref.shape[1]
    z = wm1t_ref[...] @ eet_ref[...] + bm1_ref[...]
    z = z * jax.nn.sigmoid(z)
    w = wm2t_ref[...] @ z + bm2_ref[...]
    scal_ref[pl.ds(i * blk, blk)] = jnp.sum(w * eat_ref[...], axis=0)


def _edge_tc(eet, eat, wm1t, bm1c, wm2t, bm2c):
    blk = 16000
    grid = (N_EDGES // blk,)
    s_dim = eet.shape[0]
    h_dim = wm1t.shape[0]
    return pl.pallas_call(
        _edge_body,
        grid=grid,
        in_specs=[
            pl.BlockSpec((s_dim, blk), lambda i: (0, i)),
            pl.BlockSpec((s_dim, blk), lambda i: (0, i)),
            pl.BlockSpec((h_dim, s_dim), lambda i: (0, 0)),
            pl.BlockSpec((h_dim, 1), lambda i: (0, 0)),
            pl.BlockSpec((s_dim, h_dim), lambda i: (0, 0)),
            pl.BlockSpec((s_dim, 1), lambda i: (0, 0)),
        ],
        out_specs=pl.BlockSpec((N_EDGES,), lambda i: (0,)),
        out_shape=jax.ShapeDtypeStruct((N_EDGES,), jnp.float32),
    )(eet, eat, wm1t, bm1c, wm2t, bm2c)


def _sc_body(h_hbm, src_hbm, dst_hbm, scal_hbm, out_hbm,
             src_sb, dst_sb, scal_sb, rows, agg, gsem, ssem):
    c = lax.axis_index("c")
    s = lax.axis_index("s")
    wid = c * NS + s

    zero16 = jnp.zeros((16,), jnp.float32)

    @pl.loop(0, CHUNK)
    def _zero_rows(i):
        for d in range(DIM // 16):
            rows[0, i, pl.ds(d * 16, 16)] = zero16

    base = s * ROWS_PER_TILE
    off = 0
    for cnt in (128, 128, 128, 128, 112):
        pltpu.sync_copy(rows.at[0, pl.ds(0, cnt)], agg.at[pl.ds(base + off, cnt)])
        off += cnt

    @pl.when(s == NS - 1)
    def _zero_tail():
        pltpu.sync_copy(rows.at[0, pl.ds(0, 16)],
                        agg.at[pl.ds(NS * ROWS_PER_TILE, 16)])

    row0 = wid * CH_PER_W
    plsc.subcore_barrier()

    @pl.loop(0, CH_PER_W // SUPER)
    def _super(k):
        r0 = row0 + k * SUPER
        pltpu.sync_copy(src_hbm.at[pl.ds(r0, SUPER)], src_sb)
        pltpu.sync_copy(dst_hbm.at[pl.ds(r0, SUPER)], dst_sb)
        pltpu.sync_copy(scal_hbm.at[pl.ds(r0 * CHUNK, SUPER * CHUNK)], scal_sb)

        for b in range(NBUF):
            pltpu.async_copy(h_hbm.at[src_sb.at[b]], rows.at[b], gsem.at[b])

        @pl.loop(0, SUPER // NBUF)
        def _ring(t):
            for b in range(NBUF):
                j = t * NBUF + b
                bprev = (b - 1) % NBUF
                pltpu.make_async_copy(h_hbm.at[src_sb.at[0]], rows.at[b],
                                      gsem.at[b]).wait()

                @pl.when(j > 0)
                def _retire():
                    pltpu.make_async_copy(rows.at[bprev],
                                          agg.at[dst_sb.at[0]],
                                          ssem.at[bprev]).wait()

                    @pl.when(j + NBUF - 1 < SUPER)
                    def _refill():
                        pltpu.async_copy(h_hbm.at[src_sb.at[j + NBUF - 1]],
                                         rows.at[bprev], gsem.at[bprev])

                @pl.loop(0, CHUNK, unroll=8)
                def _scale(e):
                    sv = plsc.load_gather(
                        scal_sb, [jnp.full((16,), j * CHUNK + e, jnp.int32)])
                    for d in range(DIM // 16):
                        rows[b, e, pl.ds(d * 16, 16)] = (
                            rows[b, e, pl.ds(d * 16, 16)] * sv)

                pltpu.async_copy(rows.at[b], agg.at[dst_sb.at[j]], ssem.at[b],
                                 add=True)

        pltpu.make_async_copy(rows.at[NBUF - 1], agg.at[dst_sb.at[0]],
                              ssem.at[NBUF - 1]).wait()

    plsc.subcore_barrier()
    pltpu.sync_copy(agg.at[pl.ds(base, ROWS_PER_TILE)],
                    out_hbm.at[c, pl.ds(base, ROWS_PER_TILE)])

    @pl.when(s == NS - 1)
    def _copy_tail():
        pltpu.sync_copy(agg.at[pl.ds(NS * ROWS_PER_TILE, 16)],
                        out_hbm.at[c, pl.ds(NS * ROWS_PER_TILE, 16)])


def _sc_scatter(h, src2d, dst2d, scal2d):
    mesh = plsc.VectorSubcoreMesh(core_axis_name="c", subcore_axis_name="s")
    k = pl.kernel(
        _sc_body,
        out_type=jax.ShapeDtypeStruct((NC, N_NODES, DIM), jnp.float32),
        mesh=mesh,
        scratch_types=[
            pltpu.VMEM((SUPER, CHUNK), jnp.int32),
            pltpu.VMEM((SUPER, CHUNK), jnp.int32),
            pltpu.VMEM((SUPER * CHUNK,), jnp.float32),
            pltpu.VMEM((NBUF, CHUNK, DIM), jnp.float32),
            pltpu.VMEM_SHARED((N_NODES, DIM), jnp.float32),
            pltpu.SemaphoreType.DMA((NBUF,)),
            pltpu.SemaphoreType.DMA((NBUF,)),
        ],
        compiler_params=pltpu.CompilerParams(needs_layout_passes=False),
    )
    return k(h, src2d, dst2d, scal2d)


def _out_body(agg_ref, sc_ref, w2_ref, out_ref):
    a = agg_ref[0] + agg_ref[1]
    y = a @ w2_ref[...]
    out_ref[...] = y * jax.nn.sigmoid(y) + sc_ref[...]


def _out_tc(agg, sc, w2):
    blk = 1000
    grid = (N_NODES // blk,)
    return pl.pallas_call(
        _out_body,
        grid=grid,
        in_specs=[
            pl.BlockSpec((NC, blk, DIM), lambda i: (0, i, 0)),
            pl.BlockSpec((blk, DIM), lambda i: (i, 0)),
            pl.BlockSpec((DIM, DIM), lambda i: (0, 0)),
        ],
        out_specs=pl.BlockSpec((blk, DIM), lambda i: (i, 0)),
        out_shape=jax.ShapeDtypeStruct((N_NODES, DIM), jnp.float32),
    )(agg, sc, w2)


def kernel(x, node_attrs, edge_embedding, edge_attr, edge_index,
           W1, W_sc, Wm1, bm1, Wm2, bm2, W2):
    h, sc = _node_tc(x, node_attrs, W1, W_sc)
    scal = _edge_tc(edge_embedding.T, edge_attr.T, Wm1.T, bm1[:, None],
                    Wm2.T, bm2[:, None])

    pad = E_PAD - N_EDGES
    dst = edge_index[0]
    src = edge_index[1]
    pi = jnp.arange(pad, dtype=jnp.int32) % N_NODES
    src2d = jnp.concatenate([src, pi]).reshape(E_PAD // CHUNK, CHUNK)
    dst2d = jnp.concatenate([dst, pi]).reshape(E_PAD // CHUNK, CHUNK)
    scal_p = jnp.concatenate([scal, jnp.zeros((pad,), jnp.float32)])

    agg = _sc_scatter(h, src2d, dst2d, scal_p)
    return _out_tc(agg, sc, W2)


# speedup vs baseline: 1.2422x; 1.0733x over previous
import functools
import math

import jax
import jax.numpy as jnp
from jax import lax
from jax.experimental import pallas as pl
from jax.experimental.pallas import tpu as pltpu
from jax.experimental.pallas import tpu_sc as plsc

N_NODES = 10000
N_EDGES = 320000
DIM = 128
AVG_NEIGH = 32.0

NC = 2
NS = 16
NW = NC * NS
CHUNK = 128
NBUF = 2
SUPER = 8
CH_PER_W = 80
E_PAD = NW * CH_PER_W * CHUNK
ROWS_PER_TILE = 624


def _node_body(x_ref, na_ref, w1_ref, wsc_ref, h_ref, sc_ref):
    x = x_ref[...]
    na = na_ref[...]
    h_ref[...] = (x @ w1_ref[...]) * jnp.float32(1.0 / math.sqrt(AVG_NEIGH))
    acc = (x * na[:, 0:1]) @ wsc_ref[0]
    for a in range(1, 4):
        acc = acc + (x * na[:, a:a + 1]) @ wsc_ref[a]
    sc_ref[...] = acc


def _node_tc(x, node_attrs, w1, w_sc):
    blk = 1000
    grid = (N_NODES // blk,)
    return pl.pallas_call(
        _node_body,
        grid=grid,
        in_specs=[
            pl.BlockSpec((blk, DIM), lambda i: (i, 0)),
            pl.BlockSpec((blk, 4), lambda i: (i, 0)),
            pl.BlockSpec((DIM, DIM), lambda i: (0, 0)),
            pl.BlockSpec((4, DIM, DIM), lambda i: (0, 0, 0)),
        ],
        out_specs=[
            pl.BlockSpec((blk, DIM), lambda i: (i, 0)),
            pl.BlockSpec((blk, DIM), lambda i: (i, 0)),
        ],
        out_shape=[
            jax.ShapeDtypeStruct((N_NODES, DIM), jnp.float32),
            jax.ShapeDtypeStruct((N_NODES, DIM), jnp.float32),
        ],
    )(x, node_attrs, w1, w_sc)


def _edge_body(eet_ref, eat_ref, wm1t_ref, bm1_ref, wm2t_ref, bm2_ref, scal_ref):
    i = pl.program_id(0)
    blk = eet_ref.shape[1]
    z = wm1t_ref[...] @ eet_ref[...] + bm1_ref[...]
    z = z * jax.nn.sigmoid(z)
    w = wm2t_ref[...] @ z + bm2_ref[...]
    scal_ref[pl.ds(i * blk, blk)] = jnp.sum(w * eat_ref[...], axis=0)


def _edge_tc(eet, eat, wm1t, bm1c, wm2t, bm2c):
    blk = 16000
    grid = (N_EDGES // blk,)
    s_dim = eet.shape[0]
    h_dim = wm1t.shape[0]
    return pl.pallas_call(
        _edge_body,
        grid=grid,
        in_specs=[
            pl.BlockSpec((s_dim, blk), lambda i: (0, i)),
            pl.BlockSpec((s_dim, blk), lambda i: (0, i)),
            pl.BlockSpec((h_dim, s_dim), lambda i: (0, 0)),
            pl.BlockSpec((h_dim, 1), lambda i: (0, 0)),
            pl.BlockSpec((s_dim, h_dim), lambda i: (0, 0)),
            pl.BlockSpec((s_dim, 1), lambda i: (0, 0)),
        ],
        out_specs=pl.BlockSpec((N_EDGES,), lambda i: (0,)),
        out_shape=jax.ShapeDtypeStruct((N_EDGES,), jnp.float32),
    )(eet, eat, wm1t, bm1c, wm2t, bm2c)


def _sc_body(h_hbm, ei_hbm, scal_hbm, out_hbm,
             src_sb, dst_sb, scal_sb, rows, agg, gsem, ssem, isem):
    c = lax.axis_index("c")
    s = lax.axis_index("s")
    wid = c * NS + s

    zero16 = jnp.zeros((16,), jnp.float32)

    @pl.loop(0, CHUNK)
    def _zero_rows(i):
        for d in range(DIM // 16):
            rows[0, i, pl.ds(d * 16, 16)] = zero16

    base = s * ROWS_PER_TILE
    off = 0
    for cnt in (128, 128, 128, 128, 112):
        pltpu.sync_copy(rows.at[0, pl.ds(0, cnt)], agg.at[pl.ds(base + off, cnt)])
        off += cnt

    @pl.when(s == NS - 1)
    def _zero_tail():
        pltpu.sync_copy(rows.at[0, pl.ds(0, 16)],
                        agg.at[pl.ds(NS * ROWS_PER_TILE, 16)])

    row0 = wid * CH_PER_W

    def stage(k, p, sync):
        r0 = row0 + k * SUPER
        copy = pltpu.sync_copy if sync else (
            lambda a, b: pltpu.async_copy(a, b, isem.at[p]))
        copy(ei_hbm.at[1, pl.ds(r0, SUPER)], src_sb.at[p])
        copy(ei_hbm.at[0, pl.ds(r0, SUPER)], dst_sb.at[p])
        copy(scal_hbm.at[pl.ds(r0 * CHUNK, SUPER * CHUNK)],
             scal_sb.at[pl.ds(p * SUPER * CHUNK, SUPER * CHUNK)])

    def stage_wait(p):
        pltpu.make_async_copy(ei_hbm.at[1, pl.ds(0, SUPER)], src_sb.at[p],
                              isem.at[p]).wait()
        pltpu.make_async_copy(ei_hbm.at[0, pl.ds(0, SUPER)], dst_sb.at[p],
                              isem.at[p]).wait()
        pltpu.make_async_copy(
            scal_hbm.at[pl.ds(0, SUPER * CHUNK)],
            scal_sb.at[pl.ds(p * SUPER * CHUNK, SUPER * CHUNK)],
            isem.at[p]).wait()

    stage(0, 0, True)
    for b in range(NBUF):
        pltpu.async_copy(h_hbm.at[src_sb.at[0, b]], rows.at[b], gsem.at[b])

    plsc.subcore_barrier()

    NSUP = CH_PER_W // SUPER

    @pl.loop(0, NSUP // 2)
    def _superpair(t):
        for p in range(2):
            k = 2 * t + p

            @pl.when(k + 1 < NSUP)
            def _stage_next():
                stage(k + 1, 1 - p, False)

            for j in range(SUPER):
                b = j % NBUF
                bprev = 1 - b
                pltpu.make_async_copy(h_hbm.at[src_sb.at[0, 0]],
                                      rows.at[b], gsem.at[b]).wait()

                def _retire():
                    pltpu.make_async_copy(rows.at[bprev],
                                          agg.at[dst_sb.at[0, 0]],
                                          ssem.at[bprev]).wait()
                    if j + 1 < SUPER:
                        pltpu.async_copy(h_hbm.at[src_sb.at[p, j + 1]],
                                         rows.at[bprev], gsem.at[bprev])
                    else:
                        @pl.when(k + 1 < NSUP)
                        def _refill_next_super():
                            stage_wait(1 - p)
                            pltpu.async_copy(h_hbm.at[src_sb.at[1 - p, 0]],
                                             rows.at[bprev], gsem.at[bprev])

                if j == 0:
                    pl.when(k > 0)(_retire)
                else:
                    _retire()

                @pl.loop(0, CHUNK, unroll=8)
                def _scale(e):
                    sv = plsc.load_gather(
                        scal_sb,
                        [jnp.full((16,), (p * SUPER + j) * CHUNK + e,
                                  jnp.int32)])
                    for d in range(DIM // 16):
                        rows[b, e, pl.ds(d * 16, 16)] = (
                            rows[b, e, pl.ds(d * 16, 16)] * sv)

                pltpu.async_copy(rows.at[b], agg.at[dst_sb.at[p, j]],
                                 ssem.at[b], add=True)

    pltpu.make_async_copy(rows.at[(CH_PER_W - 1) % NBUF],
                          agg.at[dst_sb.at[0, 0]],
                          ssem.at[(CH_PER_W - 1) % NBUF]).wait()

    plsc.subcore_barrier()
    pltpu.sync_copy(agg.at[pl.ds(base, ROWS_PER_TILE)],
                    out_hbm.at[c, pl.ds(base, ROWS_PER_TILE)])

    @pl.when(s == NS - 1)
    def _copy_tail():
        pltpu.sync_copy(agg.at[pl.ds(NS * ROWS_PER_TILE, 16)],
                        out_hbm.at[c, pl.ds(NS * ROWS_PER_TILE, 16)])


def _sc_scatter(h, ei3, scal_p):
    mesh = plsc.VectorSubcoreMesh(core_axis_name="c", subcore_axis_name="s")
    k = pl.kernel(
        _sc_body,
        out_type=jax.ShapeDtypeStruct((NC, N_NODES, DIM), jnp.float32),
        mesh=mesh,
        scratch_types=[
            pltpu.VMEM((2, SUPER, CHUNK), jnp.int32),
            pltpu.VMEM((2, SUPER, CHUNK), jnp.int32),
            pltpu.VMEM((2 * SUPER * CHUNK,), jnp.float32),
            pltpu.VMEM((NBUF, CHUNK, DIM), jnp.float32),
            pltpu.VMEM_SHARED((N_NODES, DIM), jnp.float32),
            pltpu.SemaphoreType.DMA((NBUF,)),
            pltpu.SemaphoreType.DMA((NBUF,)),
            pltpu.SemaphoreType.DMA((2,)),
        ],
        compiler_params=pltpu.CompilerParams(needs_layout_passes=False),
    )
    return k(h, ei3, scal_p)


def _out_body(agg_ref, sc_ref, w2_ref, out_ref):
    a = agg_ref[0] + agg_ref[1]
    y = a @ w2_ref[...]
    out_ref[...] = y * jax.nn.sigmoid(y) + sc_ref[...]


def _out_tc(agg, sc, w2):
    blk = 1000
    grid = (N_NODES // blk,)
    return pl.pallas_call(
        _out_body,
        grid=grid,
        in_specs=[
            pl.BlockSpec((NC, blk, DIM), lambda i: (0, i, 0)),
            pl.BlockSpec((blk, DIM), lambda i: (i, 0)),
            pl.BlockSpec((DIM, DIM), lambda i: (0, 0)),
        ],
        out_specs=pl.BlockSpec((blk, DIM), lambda i: (i, 0)),
        out_shape=jax.ShapeDtypeStruct((N_NODES, DIM), jnp.float32),
    )(agg, sc, w2)


def kernel(x, node_attrs, edge_embedding, edge_attr, edge_index,
           W1, W_sc, Wm1, bm1, Wm2, bm2, W2):
    h, sc = _node_tc(x, node_attrs, W1, W_sc)
    scal = _edge_tc(edge_embedding.T, edge_attr.T, Wm1.T, bm1[:, None],
                    Wm2.T, bm2[:, None])

    pad = E_PAD - N_EDGES
    pi = jnp.arange(pad, dtype=jnp.int32) % N_NODES
    ei3 = jnp.concatenate(
        [edge_index, jnp.stack([pi, pi])], axis=1
    ).reshape(2, E_PAD // CHUNK, CHUNK)
    scal_p = jnp.concatenate([scal, jnp.zeros((pad,), jnp.float32)])

    agg = _sc_scatter(h, ei3, scal_p)
    return _out_tc(agg, sc, W2)


# speedup vs baseline: 1.2569x; 1.0119x over previous
import functools
import math

import jax
import jax.numpy as jnp
from jax import lax
from jax.experimental import pallas as pl
from jax.experimental.pallas import tpu as pltpu
from jax.experimental.pallas import tpu_sc as plsc

N_NODES = 10000
N_EDGES = 320000
DIM = 128
AVG_NEIGH = 32.0

NC = 2
NS = 16
NW = NC * NS
CHUNK = 128
NBUF = 2
SUPER = 8
CH_PER_W = 80
E_PAD = NW * CH_PER_W * CHUNK
ROWS_PER_TILE = 624


def _node_body(x_ref, na_ref, w1_ref, wsc_ref, h_ref, sc_ref):
    x = x_ref[...]
    na = na_ref[...]
    h_ref[...] = (x @ w1_ref[...]) * jnp.float32(1.0 / math.sqrt(AVG_NEIGH))
    acc = (x * na[:, 0:1]) @ wsc_ref[0]
    for a in range(1, 4):
        acc = acc + (x * na[:, a:a + 1]) @ wsc_ref[a]
    sc_ref[...] = acc


def _node_tc(x, node_attrs, w1, w_sc):
    blk = 1000
    grid = (N_NODES // blk,)
    return pl.pallas_call(
        _node_body,
        grid=grid,
        in_specs=[
            pl.BlockSpec((blk, DIM), lambda i: (i, 0)),
            pl.BlockSpec((blk, 4), lambda i: (i, 0)),
            pl.BlockSpec((DIM, DIM), lambda i: (0, 0)),
            pl.BlockSpec((4, DIM, DIM), lambda i: (0, 0, 0)),
        ],
        out_specs=[
            pl.BlockSpec((blk, DIM), lambda i: (i, 0)),
            pl.BlockSpec((blk, DIM), lambda i: (i, 0)),
        ],
        out_shape=[
            jax.ShapeDtypeStruct((N_NODES, DIM), jnp.float32),
            jax.ShapeDtypeStruct((N_NODES, DIM), jnp.float32),
        ],
    )(x, node_attrs, w1, w_sc)


def _edge_body(eet_ref, eat_ref, wm1t_ref, bm1_ref, wm2t_ref, bm2_ref, scal_ref):
    i = pl.program_id(0)
    blk = eet_ref.shape[1]
    z = wm1t_ref[...] @ eet_ref[...] + bm1_ref[...]
    z = z * jax.nn.sigmoid(z)
    w = wm2t_ref[...] @ z + bm2_ref[...]
    scal_ref[pl.ds(i * blk, blk)] = jnp.sum(w * eat_ref[...], axis=0)


def _edge_tc(eet, eat, wm1t, bm1c, wm2t, bm2c):
    blk = 32000
    grid = (N_EDGES // blk,)
    s_dim = eet.shape[0]
    h_dim = wm1t.shape[0]
    return pl.pallas_call(
        _edge_body,
        grid=grid,
        in_specs=[
            pl.BlockSpec((s_dim, blk), lambda i: (0, i)),
            pl.BlockSpec((s_dim, blk), lambda i: (0, i)),
            pl.BlockSpec((h_dim, s_dim), lambda i: (0, 0)),
            pl.BlockSpec((h_dim, 1), lambda i: (0, 0)),
            pl.BlockSpec((s_dim, h_dim), lambda i: (0, 0)),
            pl.BlockSpec((s_dim, 1), lambda i: (0, 0)),
        ],
        out_specs=pl.BlockSpec((N_EDGES,), lambda i: (0,)),
        out_shape=jax.ShapeDtypeStruct((N_EDGES,), jnp.float32),
    )(eet, eat, wm1t, bm1c, wm2t, bm2c)


def _sc_body(h_hbm, ei_hbm, scal_hbm, out_hbm,
             src_sb, dst_sb, scal_sb, rows, agg, gsem, ssem, isem):
    c = lax.axis_index("c")
    s = lax.axis_index("s")
    wid = c * NS + s

    zero16 = jnp.zeros((16,), jnp.float32)

    @pl.loop(0, CHUNK)
    def _zero_rows(i):
        for d in range(DIM // 16):
            rows[0, i, pl.ds(d * 16, 16)] = zero16

    base = s * ROWS_PER_TILE
    off = 0
    for cnt in (128, 128, 128, 128, 112):
        pltpu.sync_copy(rows.at[0, pl.ds(0, cnt)], agg.at[pl.ds(base + off, cnt)])
        off += cnt

    @pl.when(s == NS - 1)
    def _zero_tail():
        pltpu.sync_copy(rows.at[0, pl.ds(0, 16)],
                        agg.at[pl.ds(NS * ROWS_PER_TILE, 16)])

    row0 = wid * CH_PER_W

    def stage(k, p, sync):
        r0 = row0 + k * SUPER
        copy = pltpu.sync_copy if sync else (
            lambda a, b: pltpu.async_copy(a, b, isem.at[p]))
        copy(ei_hbm.at[1, pl.ds(r0, SUPER)], src_sb.at[p])
        copy(ei_hbm.at[0, pl.ds(r0, SUPER)], dst_sb.at[p])
        copy(scal_hbm.at[pl.ds(r0 * CHUNK, SUPER * CHUNK)],
             scal_sb.at[pl.ds(p * SUPER * CHUNK, SUPER * CHUNK)])

    def stage_wait(p):
        pltpu.make_async_copy(ei_hbm.at[1, pl.ds(0, SUPER)], src_sb.at[p],
                              isem.at[p]).wait()
        pltpu.make_async_copy(ei_hbm.at[0, pl.ds(0, SUPER)], dst_sb.at[p],
                              isem.at[p]).wait()
        pltpu.make_async_copy(
            scal_hbm.at[pl.ds(0, SUPER * CHUNK)],
            scal_sb.at[pl.ds(p * SUPER * CHUNK, SUPER * CHUNK)],
            isem.at[p]).wait()

    stage(0, 0, True)
    for b in range(NBUF):
        pltpu.async_copy(h_hbm.at[src_sb.at[0, b]], rows.at[b], gsem.at[b])

    plsc.subcore_barrier()

    NSUP = CH_PER_W // SUPER

    @pl.loop(0, NSUP // 2)
    def _superpair(t):
        for p in range(2):
            k = 2 * t + p

            @pl.when(k + 1 < NSUP)
            def _stage_next():
                stage(k + 1, 1 - p, False)

            for j in range(SUPER):
                b = j % NBUF
                bprev = 1 - b
                pltpu.make_async_copy(h_hbm.at[src_sb.at[0, 0]],
                                      rows.at[b], gsem.at[b]).wait()

                def _retire():
                    pltpu.make_async_copy(rows.at[bprev],
                                          agg.at[dst_sb.at[0, 0]],
                                          ssem.at[bprev]).wait()
                    if j + 1 < SUPER:
                        pltpu.async_copy(h_hbm.at[src_sb.at[p, j + 1]],
                                         rows.at[bprev], gsem.at[bprev])
                    else:
                        @pl.when(k + 1 < NSUP)
                        def _refill_next_super():
                            stage_wait(1 - p)
                            pltpu.async_copy(h_hbm.at[src_sb.at[1 - p, 0]],
                                             rows.at[bprev], gsem.at[bprev])

                if j == 0:
                    pl.when(k > 0)(_retire)
                else:
                    _retire()

                @pl.loop(0, CHUNK, unroll=8)
                def _scale(e):
                    sv = plsc.load_gather(
                        scal_sb,
                        [jnp.full((16,), (p * SUPER + j) * CHUNK + e,
                                  jnp.int32)])
                    for d in range(DIM // 16):
                        rows[b, e, pl.ds(d * 16, 16)] = (
                            rows[b, e, pl.ds(d * 16, 16)] * sv)

                pltpu.async_copy(rows.at[b], agg.at[dst_sb.at[p, j]],
                                 ssem.at[b], add=True)

    pltpu.make_async_copy(rows.at[(CH_PER_W - 1) % NBUF],
                          agg.at[dst_sb.at[0, 0]],
                          ssem.at[(CH_PER_W - 1) % NBUF]).wait()

    plsc.subcore_barrier()
    pltpu.sync_copy(agg.at[pl.ds(base, ROWS_PER_TILE)],
                    out_hbm.at[c, pl.ds(base, ROWS_PER_TILE)])

    @pl.when(s == NS - 1)
    def _copy_tail():
        pltpu.sync_copy(agg.at[pl.ds(NS * ROWS_PER_TILE, 16)],
                        out_hbm.at[c, pl.ds(NS * ROWS_PER_TILE, 16)])


def _sc_scatter(h, ei3, scal_p):
    mesh = plsc.VectorSubcoreMesh(core_axis_name="c", subcore_axis_name="s")
    k = pl.kernel(
        _sc_body,
        out_type=jax.ShapeDtypeStruct((NC, N_NODES, DIM), jnp.float32),
        mesh=mesh,
        scratch_types=[
            pltpu.VMEM((2, SUPER, CHUNK), jnp.int32),
            pltpu.VMEM((2, SUPER, CHUNK), jnp.int32),
            pltpu.VMEM((2 * SUPER * CHUNK,), jnp.float32),
            pltpu.VMEM((NBUF, CHUNK, DIM), jnp.float32),
            pltpu.VMEM_SHARED((N_NODES, DIM), jnp.float32),
            pltpu.SemaphoreType.DMA((NBUF,)),
            pltpu.SemaphoreType.DMA((NBUF,)),
            pltpu.SemaphoreType.DMA((2,)),
        ],
        compiler_params=pltpu.CompilerParams(needs_layout_passes=False),
    )
    return k(h, ei3, scal_p)


def _out_body(agg_ref, sc_ref, w2_ref, out_ref):
    a = agg_ref[0] + agg_ref[1]
    y = a @ w2_ref[...]
    out_ref[...] = y * jax.nn.sigmoid(y) + sc_ref[...]


def _out_tc(agg, sc, w2):
    blk = 1000
    grid = (N_NODES // blk,)
    return pl.pallas_call(
        _out_body,
        grid=grid,
        in_specs=[
            pl.BlockSpec((NC, blk, DIM), lambda i: (0, i, 0)),
            pl.BlockSpec((blk, DIM), lambda i: (i, 0)),
            pl.BlockSpec((DIM, DIM), lambda i: (0, 0)),
        ],
        out_specs=pl.BlockSpec((blk, DIM), lambda i: (i, 0)),
        out_shape=jax.ShapeDtypeStruct((N_NODES, DIM), jnp.float32),
    )(agg, sc, w2)


def kernel(x, node_attrs, edge_embedding, edge_attr, edge_index,
           W1, W_sc, Wm1, bm1, Wm2, bm2, W2):
    h, sc = _node_tc(x, node_attrs, W1, W_sc)
    scal = _edge_tc(edge_embedding.T, edge_attr.T, Wm1.T, bm1[:, None],
                    Wm2.T, bm2[:, None])

    pad = E_PAD - N_EDGES
    pi = jnp.arange(pad, dtype=jnp.int32) % N_NODES
    ei3 = jnp.concatenate(
        [edge_index, jnp.stack([pi, pi])], axis=1
    ).reshape(2, E_PAD // CHUNK, CHUNK)
    scal_p = jnp.concatenate([scal, jnp.zeros((pad,), jnp.float32)])

    agg = _sc_scatter(h, ei3, scal_p)
    return _out_tc(agg, sc, W2)


# speedup vs baseline: 1.2805x; 1.0188x over previous
import functools
import math

import jax
import jax.numpy as jnp
from jax import lax
from jax.experimental import pallas as pl
from jax.experimental.pallas import tpu as pltpu
from jax.experimental.pallas import tpu_sc as plsc

N_NODES = 10000
N_EDGES = 320000
DIM = 128
AVG_NEIGH = 32.0

NC = 2
NS = 16
NW = NC * NS
CHUNK = 128
NBUF = 2
SUPER = 8
CH_PER_W = 80
E_PAD = NW * CH_PER_W * CHUNK
ROWS_PER_TILE = 624


def _node_body(x_ref, na_ref, w1_ref, wsc_ref, h_ref, sc_ref):
    x = x_ref[...]
    na = na_ref[...]
    h_ref[...] = (x @ w1_ref[...]) * jnp.float32(1.0 / math.sqrt(AVG_NEIGH))
    acc = (x * na[:, 0:1]) @ wsc_ref[0]
    for a in range(1, 4):
        acc = acc + (x * na[:, a:a + 1]) @ wsc_ref[a]
    sc_ref[...] = acc


def _node_tc(x, node_attrs, w1, w_sc):
    blk = 2000
    grid = (N_NODES // blk,)
    return pl.pallas_call(
        _node_body,
        grid=grid,
        in_specs=[
            pl.BlockSpec((blk, DIM), lambda i: (i, 0)),
            pl.BlockSpec((blk, 4), lambda i: (i, 0)),
            pl.BlockSpec((DIM, DIM), lambda i: (0, 0)),
            pl.BlockSpec((4, DIM, DIM), lambda i: (0, 0, 0)),
        ],
        out_specs=[
            pl.BlockSpec((blk, DIM), lambda i: (i, 0)),
            pl.BlockSpec((blk, DIM), lambda i: (i, 0)),
        ],
        out_shape=[
            jax.ShapeDtypeStruct((N_NODES, DIM), jnp.float32),
            jax.ShapeDtypeStruct((N_NODES, DIM), jnp.float32),
        ],
    )(x, node_attrs, w1, w_sc)


def _edge_body(eet_ref, eat_ref, wm1t_ref, bm1_ref, wm2t_ref, bm2_ref, scal_ref):
    i = pl.program_id(0)
    blk = eet_ref.shape[1]
    z = wm1t_ref[...] @ eet_ref[...] + bm1_ref[...]
    z = z * jax.nn.sigmoid(z)
    w = wm2t_ref[...] @ z + bm2_ref[...]
    scal_ref[pl.ds(i * blk, blk)] = jnp.sum(w * eat_ref[...], axis=0)


def _edge_tc(eet, eat, wm1t, bm1c, wm2t, bm2c):
    blk = 32000
    grid = (N_EDGES // blk,)
    s_dim = eet.shape[0]
    h_dim = wm1t.shape[0]
    return pl.pallas_call(
        _edge_body,
        grid=grid,
        in_specs=[
            pl.BlockSpec((s_dim, blk), lambda i: (0, i)),
            pl.BlockSpec((s_dim, blk), lambda i: (0, i)),
            pl.BlockSpec((h_dim, s_dim), lambda i: (0, 0)),
            pl.BlockSpec((h_dim, 1), lambda i: (0, 0)),
            pl.BlockSpec((s_dim, h_dim), lambda i: (0, 0)),
            pl.BlockSpec((s_dim, 1), lambda i: (0, 0)),
        ],
        out_specs=pl.BlockSpec((N_EDGES,), lambda i: (0,)),
        out_shape=jax.ShapeDtypeStruct((N_EDGES,), jnp.float32),
    )(eet, eat, wm1t, bm1c, wm2t, bm2c)


def _sc_body(h_hbm, ei_hbm, scal_hbm, out_hbm,
             src_sb, dst_sb, scal_sb, rows, agg, gsem, ssem, isem):
    c = lax.axis_index("c")
    s = lax.axis_index("s")
    wid = c * NS + s

    zero16 = jnp.zeros((16,), jnp.float32)

    @pl.loop(0, CHUNK)
    def _zero_rows(i):
        for d in range(DIM // 16):
            rows[0, i, pl.ds(d * 16, 16)] = zero16

    base = s * ROWS_PER_TILE
    off = 0
    for cnt in (128, 128, 128, 128, 112):
        pltpu.sync_copy(rows.at[0, pl.ds(0, cnt)], agg.at[pl.ds(base + off, cnt)])
        off += cnt

    @pl.when(s == NS - 1)
    def _zero_tail():
        pltpu.sync_copy(rows.at[0, pl.ds(0, 16)],
                        agg.at[pl.ds(NS * ROWS_PER_TILE, 16)])

    row0 = wid * CH_PER_W

    def stage(k, p, sync):
        r0 = row0 + k * SUPER
        copy = pltpu.sync_copy if sync else (
            lambda a, b: pltpu.async_copy(a, b, isem.at[p]))
        copy(ei_hbm.at[1, pl.ds(r0, SUPER)], src_sb.at[p])
        copy(ei_hbm.at[0, pl.ds(r0, SUPER)], dst_sb.at[p])
        copy(scal_hbm.at[pl.ds(r0 * CHUNK, SUPER * CHUNK)],
             scal_sb.at[pl.ds(p * SUPER * CHUNK, SUPER * CHUNK)])

    def stage_wait(p):
        pltpu.make_async_copy(ei_hbm.at[1, pl.ds(0, SUPER)], src_sb.at[p],
                              isem.at[p]).wait()
        pltpu.make_async_copy(ei_hbm.at[0, pl.ds(0, SUPER)], dst_sb.at[p],
                              isem.at[p]).wait()
        pltpu.make_async_copy(
            scal_hbm.at[pl.ds(0, SUPER * CHUNK)],
            scal_sb.at[pl.ds(p * SUPER * CHUNK, SUPER * CHUNK)],
            isem.at[p]).wait()

    stage(0, 0, True)
    for b in range(NBUF):
        pltpu.async_copy(h_hbm.at[src_sb.at[0, b]], rows.at[b], gsem.at[b])

    plsc.subcore_barrier()

    NSUP = CH_PER_W // SUPER

    @pl.loop(0, NSUP // 2)
    def _superpair(t):
        for p in range(2):
            k = 2 * t + p

            @pl.when(k + 1 < NSUP)
            def _stage_next():
                stage(k + 1, 1 - p, False)

            for j in range(SUPER):
                b = j % NBUF
                bprev = 1 - b
                pltpu.make_async_copy(h_hbm.at[src_sb.at[0, 0]],
                                      rows.at[b], gsem.at[b]).wait()

                def _retire():
                    pltpu.make_async_copy(rows.at[bprev],
                                          agg.at[dst_sb.at[0, 0]],
                                          ssem.at[bprev]).wait()
                    if j + 1 < SUPER:
                        pltpu.async_copy(h_hbm.at[src_sb.at[p, j + 1]],
                                         rows.at[bprev], gsem.at[bprev])
                    else:
                        @pl.when(k + 1 < NSUP)
                        def _refill_next_super():
                            stage_wait(1 - p)
                            pltpu.async_copy(h_hbm.at[src_sb.at[1 - p, 0]],
                                             rows.at[bprev], gsem.at[bprev])

                if j == 0:
                    pl.when(k > 0)(_retire)
                else:
                    _retire()

                @pl.loop(0, CHUNK, unroll=8)
                def _scale(e):
                    sv = plsc.load_gather(
                        scal_sb,
                        [jnp.full((16,), (p * SUPER + j) * CHUNK + e,
                                  jnp.int32)])
                    for d in range(DIM // 16):
                        rows[b, e, pl.ds(d * 16, 16)] = (
                            rows[b, e, pl.ds(d * 16, 16)] * sv)

                pltpu.async_copy(rows.at[b], agg.at[dst_sb.at[p, j]],
                                 ssem.at[b], add=True)

    pltpu.make_async_copy(rows.at[(CH_PER_W - 1) % NBUF],
                          agg.at[dst_sb.at[0, 0]],
                          ssem.at[(CH_PER_W - 1) % NBUF]).wait()

    plsc.subcore_barrier()
    pltpu.sync_copy(agg.at[pl.ds(base, ROWS_PER_TILE)],
                    out_hbm.at[c, pl.ds(base, ROWS_PER_TILE)])

    @pl.when(s == NS - 1)
    def _copy_tail():
        pltpu.sync_copy(agg.at[pl.ds(NS * ROWS_PER_TILE, 16)],
                        out_hbm.at[c, pl.ds(NS * ROWS_PER_TILE, 16)])


def _sc_scatter(h, ei3, scal_p):
    mesh = plsc.VectorSubcoreMesh(core_axis_name="c", subcore_axis_name="s")
    k = pl.kernel(
        _sc_body,
        out_type=jax.ShapeDtypeStruct((NC, N_NODES, DIM), jnp.float32),
        mesh=mesh,
        scratch_types=[
            pltpu.VMEM((2, SUPER, CHUNK), jnp.int32),
            pltpu.VMEM((2, SUPER, CHUNK), jnp.int32),
            pltpu.VMEM((2 * SUPER * CHUNK,), jnp.float32),
            pltpu.VMEM((NBUF, CHUNK, DIM), jnp.float32),
            pltpu.VMEM_SHARED((N_NODES, DIM), jnp.float32),
            pltpu.SemaphoreType.DMA((NBUF,)),
            pltpu.SemaphoreType.DMA((NBUF,)),
            pltpu.SemaphoreType.DMA((2,)),
        ],
        compiler_params=pltpu.CompilerParams(needs_layout_passes=False),
    )
    return k(h, ei3, scal_p)


def _out_body(agg_ref, sc_ref, w2_ref, out_ref):
    a = agg_ref[0] + agg_ref[1]
    y = a @ w2_ref[...]
    out_ref[...] = y * jax.nn.sigmoid(y) + sc_ref[...]


def _out_tc(agg, sc, w2):
    blk = 2000
    grid = (N_NODES // blk,)
    return pl.pallas_call(
        _out_body,
        grid=grid,
        in_specs=[
            pl.BlockSpec((NC, blk, DIM), lambda i: (0, i, 0)),
            pl.BlockSpec((blk, DIM), lambda i: (i, 0)),
            pl.BlockSpec((DIM, DIM), lambda i: (0, 0)),
        ],
        out_specs=pl.BlockSpec((blk, DIM), lambda i: (i, 0)),
        out_shape=jax.ShapeDtypeStruct((N_NODES, DIM), jnp.float32),
    )(agg, sc, w2)


def kernel(x, node_attrs, edge_embedding, edge_attr, edge_index,
           W1, W_sc, Wm1, bm1, Wm2, bm2, W2):
    h, sc = _node_tc(x, node_attrs, W1, W_sc)
    scal = _edge_tc(edge_embedding.T, edge_attr.T, Wm1.T, bm1[:, None],
                    Wm2.T, bm2[:, None])

    pad = E_PAD - N_EDGES
    pi = jnp.arange(pad, dtype=jnp.int32) % N_NODES
    ei3 = jnp.concatenate(
        [edge_index, jnp.stack([pi, pi])], axis=1
    ).reshape(2, E_PAD // CHUNK, CHUNK)
    scal_p = jnp.concatenate([scal, jnp.zeros((pad,), jnp.float32)])

    agg = _sc_scatter(h, ei3, scal_p)
    return _out_tc(agg, sc, W2)
